# Initial kernel scaffold; baseline (speedup 1.0000x reference)
import functools

import jax
import jax.numpy as jnp
from jax import lax
from jax.experimental import pallas as pl
from jax.experimental.pallas import tpu as pltpu
from jax.experimental.pallas import tpu_sc as plsc

NC = 2
NS = 16
L = 16

CHUNK = 256
GROUPS = CHUNK // L


def _router_body(logits_hbm, out_hbm, in_v, out_v, idx_v):
    B = logits_hbm.shape[0]
    E = logits_hbm.shape[1]
    rows_per_worker = B // (NC * NS)
    chunks = rows_per_worker // CHUNK

    wid = lax.axis_index("s") * NC + lax.axis_index("c")
    wbase = wid * rows_per_worker

    iota = lax.iota(jnp.int32, L)
    zeros = jnp.zeros((L,), jnp.float32)
    neg_inf = jnp.full((L,), -jnp.inf, jnp.float32)
    izero = jnp.zeros((L,), jnp.int32)

    def _zero_row(r, c):
        for q in range(E // L):
            out_v[r, pl.ds(q * L, L)] = zeros
        return c

    lax.fori_loop(0, CHUNK, _zero_row, 0)

    def _col_body(j, carry):
        m1, m2, i1, i2, row_idx = carry
        jv = izero + j
        v = plsc.load_gather(in_v, [row_idx, jv])
        gt1 = v > m1
        gt2 = v > m2
        m2 = jnp.where(gt1, m1, jnp.where(gt2, v, m2))
        i2 = jnp.where(gt1, i1, jnp.where(gt2, jv, i2))
        m1 = jnp.where(gt1, v, m1)
        i1 = jnp.where(gt1, jv, i1)
        return m1, m2, i1, i2, row_idx

    def _group_body(g, c):
        row_idx = g * L + iota
        m1, m2, i1, i2, _ = lax.fori_loop(
            0, E, _col_body, (neg_inf, neg_inf, izero, izero, row_idx)
        )
        e = jnp.exp(m2 - m1)
        s = 1.0 + e
        w1 = 1.0 / s
        w2 = e / s
        plsc.store_scatter(out_v, [row_idx, i1], w1)
        plsc.store_scatter(out_v, [row_idx, i2], w2)
        idx_v[g, 0, :] = i1
        idx_v[g, 1, :] = i2
        return c

    def _rezero_body(g, c):
        row_idx = g * L + iota
        i1 = idx_v[g, 0, :]
        i2 = idx_v[g, 1, :]
        plsc.store_scatter(out_v, [row_idx, i1], zeros)
        plsc.store_scatter(out_v, [row_idx, i2], zeros)
        return c

    for chunk in range(chunks):
        rbase = wbase + chunk * CHUNK
        pltpu.sync_copy(logits_hbm.at[pl.ds(rbase, CHUNK)], in_v)
        lax.fori_loop(0, GROUPS, _group_body, 0)
        pltpu.sync_copy(out_v, out_hbm.at[pl.ds(rbase, CHUNK)])
        lax.fori_loop(0, GROUPS, _rezero_body, 0)


@functools.partial(jax.jit, static_argnames=("interpret",))
def _router(logits, interpret=False):
    B, E = logits.shape
    k = pl.kernel(
        _router_body,
        out_type=jax.ShapeDtypeStruct((B, E), jnp.float32),
        mesh=plsc.VectorSubcoreMesh(
            core_axis_name="c", subcore_axis_name="s"
        ),
        scratch_types=[
            pltpu.VMEM((CHUNK, E), jnp.float32),
            pltpu.VMEM((CHUNK, E), jnp.float32),
            pltpu.VMEM((GROUPS, 2, L), jnp.int32),
        ],
        interpret=interpret,
    )
    return k(logits)


def kernel(logits):
    return _router(logits)


# speedup vs baseline: 3.2653x; 3.2653x over previous
import functools

import jax
import jax.numpy as jnp
from jax import lax
from jax.experimental import pallas as pl
from jax.experimental.pallas import tpu as pltpu
from jax.experimental.pallas import tpu_sc as plsc

NC = 2
NS = 16
L = 16

CHUNK = 256
GROUPS = CHUNK // L

B = 32768
E = 64


def _router_body(logits_hbm, out_hbm, in_v, out_v, idx_v):
    rows_per_worker = B // (NC * NS)
    chunks = rows_per_worker // CHUNK

    wid = lax.axis_index("s") * NC + lax.axis_index("c")
    wbase = wid * rows_per_worker

    iota = lax.iota(jnp.int32, L)
    zeros = jnp.zeros((L,), jnp.float32)
    neg_inf = jnp.full((L,), -jnp.inf, jnp.float32)
    izero = jnp.zeros((L,), jnp.int32)

    def _zero_row(r, c):
        out_v[pl.ds(r * L, L)] = zeros
        return c

    lax.fori_loop(0, CHUNK * E // L, _zero_row, 0)

    def _col_body(j, carry):
        m1, m2, i1, i2, flat_base = carry
        jv = izero + j
        v = plsc.load_gather(in_v, [flat_base + j])
        gt1 = v > m1
        gt2 = v > m2
        m2 = jnp.where(gt1, m1, jnp.where(gt2, v, m2))
        i2 = jnp.where(gt1, i1, jnp.where(gt2, jv, i2))
        m1 = jnp.where(gt1, v, m1)
        i1 = jnp.where(gt1, jv, i1)
        return m1, m2, i1, i2, flat_base

    def _group_body(g, c):
        flat_base = (g * L + iota) * E
        m1, m2, i1, i2, _ = lax.fori_loop(
            0, E, _col_body, (neg_inf, neg_inf, izero, izero, flat_base)
        )
        e = jnp.exp(m2 - m1)
        s = 1.0 + e
        w1 = 1.0 / s
        w2 = e / s
        plsc.store_scatter(out_v, [flat_base + i1], w1)
        plsc.store_scatter(out_v, [flat_base + i2], w2)
        idx_v[pl.ds(g * 2 * L, L)] = i1
        idx_v[pl.ds(g * 2 * L + L, L)] = i2
        return c

    def _rezero_body(g, c):
        flat_base = (g * L + iota) * E
        i1 = idx_v[pl.ds(g * 2 * L, L)]
        i2 = idx_v[pl.ds(g * 2 * L + L, L)]
        plsc.store_scatter(out_v, [flat_base + i1], zeros)
        plsc.store_scatter(out_v, [flat_base + i2], zeros)
        return c

    for chunk in range(chunks):
        fbase = (wbase + chunk * CHUNK) * E
        pltpu.sync_copy(logits_hbm.at[pl.ds(fbase, CHUNK * E)], in_v)
        lax.fori_loop(0, GROUPS, _group_body, 0)
        pltpu.sync_copy(out_v, out_hbm.at[pl.ds(fbase, CHUNK * E)])
        lax.fori_loop(0, GROUPS, _rezero_body, 0)


@jax.jit
def _router(logits_flat):
    k = pl.kernel(
        _router_body,
        out_type=jax.ShapeDtypeStruct((B * E,), jnp.float32),
        mesh=plsc.VectorSubcoreMesh(
            core_axis_name="c", subcore_axis_name="s",
            num_cores=NC, num_subcores=NS,
        ),
        scratch_types=[
            pltpu.VMEM((CHUNK * E,), jnp.float32),
            pltpu.VMEM((CHUNK * E,), jnp.float32),
            pltpu.VMEM((GROUPS * 2 * L,), jnp.int32),
        ],
        compiler_params=pltpu.CompilerParams(needs_layout_passes=False),
    )
    return k(logits_flat)


def kernel(logits):
    return _router(logits.reshape(-1)).reshape(B, E)


# speedup vs baseline: 3.9021x; 1.1950x over previous
import jax
import jax.numpy as jnp
from jax import lax
from jax.experimental import pallas as pl
from jax.experimental.pallas import tpu as pltpu
from jax.experimental.pallas import tpu_sc as plsc

NC = 2
NS = 16
L = 16

CHUNK = 256
GROUPS = CHUNK // L
ACC = 4

B = 32768
E = 64


def _merge(a, b):
    am1, ai1, am2, ai2 = a
    bm1, bi1, bm2, bi2 = b
    agt = am1 > bm1
    aeq = am1 == bm1
    atake = agt | (aeq & (ai1 < bi1))
    m1 = jnp.where(atake, am1, bm1)
    i1 = jnp.where(atake, ai1, bi1)
    x = jnp.where(atake, bm1, am1)
    xi = jnp.where(atake, bi1, ai1)
    ygt = am2 > bm2
    yeq = am2 == bm2
    ytake = ygt | (yeq & (ai2 < bi2))
    y = jnp.where(ytake, am2, bm2)
    yi = jnp.where(ytake, ai2, bi2)
    xgt = x > y
    xeq = x == y
    xtake = xgt | (xeq & (xi < yi))
    m2 = jnp.where(xtake, x, y)
    i2 = jnp.where(xtake, xi, yi)
    return m1, i1, m2, i2


def _router_body(logits_hbm, out_hbm, in_v, out_v, idx_v):
    rows_per_worker = B // (NC * NS)
    chunks = rows_per_worker // CHUNK

    wid = lax.axis_index("s") * NC + lax.axis_index("c")
    wbase = wid * rows_per_worker

    iota = lax.iota(jnp.int32, L)
    zeros = jnp.zeros((L,), jnp.float32)
    neg_inf = jnp.full((L,), -jnp.inf, jnp.float32)
    izero = jnp.zeros((L,), jnp.int32)
    ione = jnp.full((L,), 1, jnp.int32)

    def _zero_row(r, c):
        out_v[pl.ds(r * L, L)] = zeros
        return c

    lax.fori_loop(0, CHUNK * E // L, _zero_row, 0)

    cols_per_acc = E // ACC

    def _group_body(g, c):
        flat_base = (g * L + iota) * E
        m1 = [neg_inf] * ACC
        m2 = [neg_inf] * ACC
        fi1 = [izero] * ACC
        fi2 = [izero] * ACC
        fidx = [flat_base + (q * cols_per_acc) for q in range(ACC)]
        for _ in range(cols_per_acc):
            for q in range(ACC):
                v = plsc.load_gather(in_v, [fidx[q]])
                gt1 = v > m1[q]
                gt2 = v > m2[q]
                m2[q] = jnp.where(gt1, m1[q], jnp.where(gt2, v, m2[q]))
                fi2[q] = jnp.where(gt1, fi1[q], jnp.where(gt2, fidx[q], fi2[q]))
                m1[q] = jnp.where(gt1, v, m1[q])
                fi1[q] = jnp.where(gt1, fidx[q], fi1[q])
                fidx[q] = fidx[q] + ione
        s01 = _merge((m1[0], fi1[0], m2[0], fi2[0]),
                     (m1[1], fi1[1], m2[1], fi2[1]))
        s23 = _merge((m1[2], fi1[2], m2[2], fi2[2]),
                     (m1[3], fi1[3], m2[3], fi2[3]))
        tm1, ti1, tm2, ti2 = _merge(s01, s23)

        e = jnp.exp(tm2 - tm1)
        s = 1.0 + e
        w1 = 1.0 / s
        w2 = e / s
        plsc.store_scatter(out_v, [ti1], w1)
        plsc.store_scatter(out_v, [ti2], w2)
        idx_v[pl.ds(g * 2 * L, L)] = ti1
        idx_v[pl.ds(g * 2 * L + L, L)] = ti2
        return c

    def _rezero_body(g, c):
        i1 = idx_v[pl.ds(g * 2 * L, L)]
        i2 = idx_v[pl.ds(g * 2 * L + L, L)]
        plsc.store_scatter(out_v, [i1], zeros)
        plsc.store_scatter(out_v, [i2], zeros)
        return c

    for chunk in range(chunks):
        fbase = (wbase + chunk * CHUNK) * E
        pltpu.sync_copy(logits_hbm.at[pl.ds(fbase, CHUNK * E)], in_v)
        lax.fori_loop(0, GROUPS, _group_body, 0)
        pltpu.sync_copy(out_v, out_hbm.at[pl.ds(fbase, CHUNK * E)])
        lax.fori_loop(0, GROUPS, _rezero_body, 0)


@jax.jit
def _router(logits_flat):
    k = pl.kernel(
        _router_body,
        out_type=jax.ShapeDtypeStruct((B * E,), jnp.float32),
        mesh=plsc.VectorSubcoreMesh(
            core_axis_name="c", subcore_axis_name="s",
            num_cores=NC, num_subcores=NS,
        ),
        scratch_types=[
            pltpu.VMEM((CHUNK * E,), jnp.float32),
            pltpu.VMEM((CHUNK * E,), jnp.float32),
            pltpu.VMEM((GROUPS * 2 * L,), jnp.int32),
        ],
        compiler_params=pltpu.CompilerParams(needs_layout_passes=False),
    )
    return k(logits_flat)


def kernel(logits):
    return _router(logits.reshape(-1)).reshape(B, E)


# speedup vs baseline: 4.1140x; 1.0543x over previous
import jax
import jax.numpy as jnp
from jax import lax
from jax.experimental import pallas as pl
from jax.experimental.pallas import tpu as pltpu
from jax.experimental.pallas import tpu_sc as plsc

NC = 2
NS = 16
L = 16

CHUNK = 256
GROUPS = CHUNK // L
ACC = 4

B = 32768
E = 64
PADE = E + 1


def _merge(a, b):
    am1, ai1, am2, ai2 = a
    bm1, bi1, bm2, bi2 = b
    agt = am1 > bm1
    aeq = am1 == bm1
    atake = agt | (aeq & (ai1 < bi1))
    m1 = jnp.where(atake, am1, bm1)
    i1 = jnp.where(atake, ai1, bi1)
    x = jnp.where(atake, bm1, am1)
    xi = jnp.where(atake, bi1, ai1)
    ygt = am2 > bm2
    yeq = am2 == bm2
    ytake = ygt | (yeq & (ai2 < bi2))
    y = jnp.where(ytake, am2, bm2)
    yi = jnp.where(ytake, ai2, bi2)
    xgt = x > y
    xeq = x == y
    xtake = xgt | (xeq & (xi < yi))
    m2 = jnp.where(xtake, x, y)
    i2 = jnp.where(xtake, xi, yi)
    return m1, i1, m2, i2


def _router_body(logits_hbm, out_hbm, raw_v, pad_v, out_v, idx_v):
    rows_per_worker = B // (NC * NS)
    chunks = rows_per_worker // CHUNK

    wid = lax.axis_index("s") * NC + lax.axis_index("c")
    wbase = wid * rows_per_worker

    iota = lax.iota(jnp.int32, L)
    zeros = jnp.zeros((L,), jnp.float32)
    neg_inf = jnp.full((L,), -jnp.inf, jnp.float32)
    izero = jnp.zeros((L,), jnp.int32)
    ione = jnp.full((L,), 1, jnp.int32)
    colc = [iota + (q * L) for q in range(ACC)]

    def _zero_row(r, c):
        out_v[pl.ds(r * L, L)] = zeros
        return c

    lax.fori_loop(0, CHUNK * E // L, _zero_row, 0)

    def _relayout(i, c):
        for rr in range(4):
            r = i * 4 + rr
            src = r * E
            dbase = izero + r * PADE
            for q in range(ACC):
                v = raw_v[pl.ds(src + q * L, L)]
                plsc.store_scatter(pad_v, [dbase + colc[q]], v)
        return c

    cols_per_acc = E // ACC

    def _group_body(g, c):
        row = g * L + iota
        flat_base = row * PADE
        m1 = [neg_inf] * ACC
        m2 = [neg_inf] * ACC
        fi1 = [izero] * ACC
        fi2 = [izero] * ACC
        fidx = [flat_base + (q * cols_per_acc) for q in range(ACC)]
        for _ in range(cols_per_acc):
            for q in range(ACC):
                v = plsc.load_gather(pad_v, [fidx[q]])
                gt1 = v > m1[q]
                gt2 = v > m2[q]
                m2[q] = jnp.where(gt1, m1[q], jnp.where(gt2, v, m2[q]))
                fi2[q] = jnp.where(gt1, fi1[q], jnp.where(gt2, fidx[q], fi2[q]))
                m1[q] = jnp.where(gt1, v, m1[q])
                fi1[q] = jnp.where(gt1, fidx[q], fi1[q])
                fidx[q] = fidx[q] + ione
        s01 = _merge((m1[0], fi1[0], m2[0], fi2[0]),
                     (m1[1], fi1[1], m2[1], fi2[1]))
        s23 = _merge((m1[2], fi1[2], m2[2], fi2[2]),
                     (m1[3], fi1[3], m2[3], fi2[3]))
        tm1, ti1, tm2, ti2 = _merge(s01, s23)

        e = jnp.exp(tm2 - tm1)
        s = 1.0 + e
        w1 = 1.0 / s
        w2 = e / s
        oi1 = ti1 - row
        oi2 = ti2 - row
        plsc.store_scatter(out_v, [oi1], w1)
        plsc.store_scatter(out_v, [oi2], w2)
        idx_v[pl.ds(g * 2 * L, L)] = oi1
        idx_v[pl.ds(g * 2 * L + L, L)] = oi2
        return c

    def _rezero_body(g, c):
        i1 = idx_v[pl.ds(g * 2 * L, L)]
        i2 = idx_v[pl.ds(g * 2 * L + L, L)]
        plsc.store_scatter(out_v, [i1], zeros)
        plsc.store_scatter(out_v, [i2], zeros)
        return c

    for chunk in range(chunks):
        fbase = (wbase + chunk * CHUNK) * E
        pltpu.sync_copy(logits_hbm.at[pl.ds(fbase, CHUNK * E)], raw_v)
        lax.fori_loop(0, CHUNK // 4, _relayout, 0)
        lax.fori_loop(0, GROUPS, _group_body, 0)
        pltpu.sync_copy(out_v, out_hbm.at[pl.ds(fbase, CHUNK * E)])
        lax.fori_loop(0, GROUPS, _rezero_body, 0)


@jax.jit
def _router(logits_flat):
    k = pl.kernel(
        _router_body,
        out_type=jax.ShapeDtypeStruct((B * E,), jnp.float32),
        mesh=plsc.VectorSubcoreMesh(
            core_axis_name="c", subcore_axis_name="s",
            num_cores=NC, num_subcores=NS,
        ),
        scratch_types=[
            pltpu.VMEM((CHUNK * E,), jnp.float32),
            pltpu.VMEM((CHUNK * PADE,), jnp.float32),
            pltpu.VMEM((CHUNK * E,), jnp.float32),
            pltpu.VMEM((GROUPS * 2 * L,), jnp.int32),
        ],
        compiler_params=pltpu.CompilerParams(needs_layout_passes=False),
    )
    return k(logits_flat)


def kernel(logits):
    return _router(logits.reshape(-1)).reshape(B, E)


# speedup vs baseline: 4.7294x; 1.1496x over previous
import jax
import jax.numpy as jnp
from jax import lax
from jax.experimental import pallas as pl
from jax.experimental.pallas import tpu as pltpu
from jax.experimental.pallas import tpu_sc as plsc

NC = 2
NS = 16
L = 16

CHUNK = 256
GROUPS = CHUNK // L
ACC = 4

B = 32768
E = 64


def _merge(a, b):
    am1, ai1, am2, ai2 = a
    bm1, bi1, bm2, bi2 = b
    atake = (am1 > bm1) | ((am1 == bm1) & (ai1 < bi1))
    m1 = jnp.where(atake, am1, bm1)
    i1 = jnp.where(atake, ai1, bi1)
    x = jnp.where(atake, bm1, am1)
    xi = jnp.where(atake, bi1, ai1)
    ytake = (am2 > bm2) | ((am2 == bm2) & (ai2 < bi2))
    y = jnp.where(ytake, am2, bm2)
    yi = jnp.where(ytake, ai2, bi2)
    xtake = (x > y) | ((x == y) & (xi < yi))
    m2 = jnp.where(xtake, x, y)
    i2 = jnp.where(xtake, xi, yi)
    return m1, i1, m2, i2


def _router_body(logits_hbm, out_hbm, raw_v, out_v, idx_v):
    rows_per_worker = B // (NC * NS)
    chunks = rows_per_worker // CHUNK

    wid = lax.axis_index("s") * NC + lax.axis_index("c")
    wbase = wid * rows_per_worker

    iota = lax.iota(jnp.int32, L)
    zeros = jnp.zeros((L,), jnp.float32)
    neg_inf = jnp.full((L,), -jnp.inf, jnp.float32)
    izero = jnp.zeros((L,), jnp.int32)
    ione = jnp.full((L,), 1, jnp.int32)

    cols_per_acc = E // ACC

    def _zero_row(r, c):
        for q in range(ACC):
            out_v[r, pl.ds(q * L, L)] = zeros
        return c

    lax.fori_loop(0, CHUNK, _zero_row, 0)

    def _group_body(g, c):
        row = g * L + iota
        m1 = [neg_inf] * ACC
        m2 = [neg_inf] * ACC
        i1 = [izero] * ACC
        i2 = [izero] * ACC
        cidx = [izero + (q * cols_per_acc) for q in range(ACC)]
        for _t in range(cols_per_acc):
            for q in range(ACC):
                v = plsc.load_gather(raw_v, [row, cidx[q]])
                gt1 = v > m1[q]
                gt2 = v > m2[q]
                m2[q] = jnp.where(gt1, m1[q], jnp.where(gt2, v, m2[q]))
                i2[q] = jnp.where(gt1, i1[q], jnp.where(gt2, cidx[q], i2[q]))
                m1[q] = jnp.where(gt1, v, m1[q])
                i1[q] = jnp.where(gt1, cidx[q], i1[q])
                cidx[q] = cidx[q] + ione
        s01 = _merge((m1[0], i1[0], m2[0], i2[0]),
                     (m1[1], i1[1], m2[1], i2[1]))
        s23 = _merge((m1[2], i1[2], m2[2], i2[2]),
                     (m1[3], i1[3], m2[3], i2[3]))
        tm1, ti1, tm2, ti2 = _merge(s01, s23)

        e = jnp.exp(tm2 - tm1)
        s = 1.0 + e
        w1 = 1.0 / s
        w2 = e / s
        plsc.store_scatter(out_v, [row, ti1], w1)
        plsc.store_scatter(out_v, [row, ti2], w2)
        idx_v[pl.ds(g * 2 * L, L)] = ti1
        idx_v[pl.ds(g * 2 * L + L, L)] = ti2
        return c

    def _rezero_body(g, c):
        row = g * L + iota
        c1 = idx_v[pl.ds(g * 2 * L, L)]
        c2 = idx_v[pl.ds(g * 2 * L + L, L)]
        plsc.store_scatter(out_v, [row, c1], zeros)
        plsc.store_scatter(out_v, [row, c2], zeros)
        return c

    for chunk in range(chunks):
        rbase = wbase + chunk * CHUNK
        pltpu.sync_copy(logits_hbm.at[pl.ds(rbase, CHUNK)], raw_v)
        lax.fori_loop(0, GROUPS, _group_body, 0)
        pltpu.sync_copy(out_v, out_hbm.at[pl.ds(rbase, CHUNK)])
        lax.fori_loop(0, GROUPS, _rezero_body, 0)


@jax.jit
def _router(logits):
    k = pl.kernel(
        _router_body,
        out_type=jax.ShapeDtypeStruct((B, E), jnp.float32),
        mesh=plsc.VectorSubcoreMesh(
            core_axis_name="c", subcore_axis_name="s",
            num_cores=NC, num_subcores=NS,
        ),
        scratch_types=[
            pltpu.VMEM((CHUNK, E), jnp.float32),
            pltpu.VMEM((CHUNK, E), jnp.float32),
            pltpu.VMEM((GROUPS * 2 * L,), jnp.int32),
        ],
        compiler_params=pltpu.CompilerParams(needs_layout_passes=False),
    )
    return k(logits)


def kernel(logits):
    return _router(logits)
